# baseline (speedup 1.0000x reference)
import functools

import jax
import jax.numpy as jnp
from jax import lax
from jax.experimental import pallas as pl
from jax.experimental.pallas import tpu as pltpu
from jax.experimental.pallas import tpu_sc as plsc

BATCH = 4096
SEQ = 200
FEAT = 128
NUM_ALT = 4

_B_BLK = 128
_S_BLK = 200


@functools.cache
def _make_sc_gather():
    info = plsc.get_sparse_core_info()
    nc, ns = info.num_cores, info.num_subcores
    b_per_w = BATCH // (nc * ns)

    def body(gamma_hbm, beta_hbm, idx_hbm, g_out, b_out,
             idx_v, grow_v, brow_v, sem_g, sem_b):
        wid = lax.axis_index("s") * nc + lax.axis_index("c")
        base = wid * b_per_w
        pltpu.sync_copy(idx_hbm.at[pl.ds(base, b_per_w)], idx_v)
        cp_g = pltpu.async_copy(gamma_hbm.at[idx_v], grow_v, sem_g)
        cp_b = pltpu.async_copy(beta_hbm.at[idx_v], brow_v, sem_b)
        cp_g.wait()
        cp_b.wait()
        pltpu.sync_copy(grow_v, g_out.at[pl.ds(base, b_per_w)])
        pltpu.sync_copy(brow_v, b_out.at[pl.ds(base, b_per_w)])

    return pl.kernel(
        body,
        mesh=plsc.VectorSubcoreMesh(core_axis_name="c", subcore_axis_name="s"),
        out_type=[
            jax.ShapeDtypeStruct((BATCH, FEAT), jnp.float32),
            jax.ShapeDtypeStruct((BATCH, FEAT), jnp.float32),
        ],
        scratch_types=[
            pltpu.VMEM((b_per_w,), jnp.int32),
            pltpu.VMEM((b_per_w, FEAT), jnp.float32),
            pltpu.VMEM((b_per_w, FEAT), jnp.float32),
            pltpu.SemaphoreType.DMA,
            pltpu.SemaphoreType.DMA,
        ],
    )


def _film_onehot_body(idx_ref, gamma_ref, beta_ref, feat_ref, out_ref):
    idx = idx_ref[...]
    iota = lax.broadcasted_iota(jnp.int32, (idx.shape[0], 8), 1).astype(jnp.float32)
    onehot = (idx == iota).astype(jnp.float32)
    g = jnp.dot(onehot, gamma_ref[...], preferred_element_type=jnp.float32)
    b = jnp.dot(onehot, beta_ref[...], preferred_element_type=jnp.float32)
    out_ref[...] = feat_ref[...] * g[:, None, :] + b[:, None, :]


def kernel(feat, alt_idx, gamma, beta):
    idx_f = alt_idx.astype(jnp.float32)[:, None]
    pad = jnp.zeros((8 - NUM_ALT, FEAT), jnp.float32)
    gamma8 = jnp.concatenate([gamma, pad], axis=0)
    beta8 = jnp.concatenate([beta, pad], axis=0)
    return pl.pallas_call(
        _film_onehot_body,
        grid=(BATCH // _B_BLK,),
        in_specs=[
            pl.BlockSpec((_B_BLK, 1), lambda i: (i, 0)),
            pl.BlockSpec((8, FEAT), lambda i: (0, 0)),
            pl.BlockSpec((8, FEAT), lambda i: (0, 0)),
            pl.BlockSpec((_B_BLK, SEQ, FEAT), lambda i: (i, 0, 0)),
        ],
        out_specs=pl.BlockSpec((_B_BLK, SEQ, FEAT), lambda i: (i, 0, 0)),
        out_shape=jax.ShapeDtypeStruct((BATCH, SEQ, FEAT), jnp.float32),
        compiler_params=pltpu.CompilerParams(
            vmem_limit_bytes=128 * 1024 * 1024,
        ),
    )(idx_f, gamma8, beta8, feat)


# speedup vs baseline: 1.0007x; 1.0007x over previous
import jax
import jax.numpy as jnp
from jax import lax
from jax.experimental import pallas as pl
from jax.experimental.pallas import tpu as pltpu

BATCH = 4096
SEQ = 200
FEAT = 128
NUM_ALT = 4

_B_BLK = 128


def _film_onehot_body(idx_ref, gamma_ref, beta_ref, feat_ref, out_ref):
    idx = idx_ref[...]
    iota = lax.broadcasted_iota(jnp.int32, (idx.shape[0], 8), 1).astype(jnp.float32)
    onehot = (idx == iota).astype(jnp.float32)
    g = jnp.dot(onehot, gamma_ref[...], preferred_element_type=jnp.float32)
    b = jnp.dot(onehot, beta_ref[...], preferred_element_type=jnp.float32)
    out_ref[...] = feat_ref[...] * g[:, None, :] + b[:, None, :]


def kernel(feat, alt_idx, gamma, beta):
    idx_f = alt_idx.astype(jnp.float32)[:, None]
    pad = jnp.zeros((8 - NUM_ALT, FEAT), jnp.float32)
    gamma8 = jnp.concatenate([gamma, pad], axis=0)
    beta8 = jnp.concatenate([beta, pad], axis=0)
    return pl.pallas_call(
        _film_onehot_body,
        grid=(BATCH // _B_BLK,),
        in_specs=[
            pl.BlockSpec((_B_BLK, 1), lambda i: (i, 0)),
            pl.BlockSpec((8, FEAT), lambda i: (0, 0)),
            pl.BlockSpec((8, FEAT), lambda i: (0, 0)),
            pl.BlockSpec((_B_BLK, SEQ, FEAT), lambda i: (i, 0, 0)),
        ],
        out_specs=pl.BlockSpec((_B_BLK, SEQ, FEAT), lambda i: (i, 0, 0)),
        out_shape=jax.ShapeDtypeStruct((BATCH, SEQ, FEAT), jnp.float32),
        compiler_params=pltpu.CompilerParams(
            vmem_limit_bytes=128 * 1024 * 1024,
        ),
    )(idx_f, gamma8, beta8, feat)
